# baseline (speedup 1.0000x reference)
import functools

import jax
import jax.numpy as jnp
from jax import lax
from jax.experimental import pallas as pl
from jax.experimental.pallas import tpu as pltpu
from jax.experimental.pallas import tpu_sc as plsc

NUM_DIM = 64
BATCH = 16384
NUM_PAIRS = 3
PAIR_DIM = 2 * NUM_DIM
TABLE_ROWS = 7 * 24 + 2 * 100 + 12 * 31
_REPLICAS = 6

_NC = 2
_NS = 16
_NW = _NC * _NS
_S_PER_W = BATCH // _NW
_CHUNK = 128
_NBLOCK = _S_PER_W // _CHUNK
_IDX_PER_W = _S_PER_W * NUM_PAIRS

_mesh = plsc.VectorSubcoreMesh(core_axis_name="c", subcore_axis_name="s")


@functools.partial(
    pl.kernel,
    out_type=jax.ShapeDtypeStruct((BATCH, NUM_PAIRS * PAIR_DIM), jnp.float32),
    mesh=_mesh,
    scratch_types=[
        pltpu.VMEM((6 * _S_PER_W,), jnp.int32),
        pltpu.VMEM((_IDX_PER_W,), jnp.int32),
        pltpu.VMEM((_CHUNK, NUM_PAIRS * PAIR_DIM), jnp.float32),
        pltpu.VMEM((_CHUNK, NUM_PAIRS * PAIR_DIM), jnp.float32),
        pltpu.SemaphoreType.DMA,
        pltpu.SemaphoreType.DMA,
    ],
)
def _embed_concat(i0, i1, i2, i3, i4, i5, table_hbm, out, raw_v, idx_v,
                  buf_a, buf_b, gsem, osem):
    bufs = (buf_a, buf_b)
    sid = lax.axis_index("s")
    wid = sid * _NC + lax.axis_index("c")
    base = wid * _S_PER_W

    raw_loads = [
        pltpu.async_copy(ref.at[pl.ds(base, _S_PER_W)],
                         raw_v.at[pl.ds(f * _S_PER_W, _S_PER_W)], osem)
        for f, ref in enumerate((i0, i1, i2, i3, i4, i5))
    ]
    for ld in raw_loads:
        ld.wait()

    def build_block(b):
        for c, (mult, offset) in enumerate(((24, 0), (100, 168), (31, 368))):
            bias = offset + ((wid * _NBLOCK * NUM_PAIRS
                              + b * NUM_PAIRS + c) % _REPLICAS) * TABLE_ROWS
            for k in range(_CHUNK // 16):
                s = b * _CHUNK + k * 16
                va = raw_v[pl.ds(2 * c * _S_PER_W + s, 16)]
                vb = raw_v[pl.ds((2 * c + 1) * _S_PER_W + s, 16)]
                idx_v[pl.ds((b * NUM_PAIRS + c) * _CHUNK + k * 16, 16)] = (
                    va * mult + vb + bias)

    build_block(0)
    out_writes = []
    for b in range(_NBLOCK):
        buf = bufs[b % 2]
        if b >= 2:
            out_writes[b - 2].wait()
        gathers = [
            pltpu.async_copy(
                table_hbm.at[idx_v.at[pl.ds((b * NUM_PAIRS + c) * _CHUNK,
                                           _CHUNK)]],
                buf.at[:, pl.ds(c * PAIR_DIM, PAIR_DIM)],
                gsem,
            )
            for c in range(NUM_PAIRS)
        ]
        if b + 1 < _NBLOCK:
            build_block(b + 1)
        for g in gathers:
            g.wait()
        out_writes.append(
            pltpu.async_copy(
                buf,
                out.at[pl.ds(wid * _S_PER_W + b * _CHUNK, _CHUNK)],
                osem,
            )
        )
    out_writes[_NBLOCK - 2].wait()
    out_writes[_NBLOCK - 1].wait()


def _pair_table(wa, wb):
    va, vb = wa.shape[0], wb.shape[0]
    return jnp.concatenate(
        [jnp.broadcast_to(wa[:, None, :], (va, vb, NUM_DIM)),
         jnp.broadcast_to(wb[None, :, :], (va, vb, NUM_DIM))],
        axis=-1,
    ).reshape(va * vb, PAIR_DIM)


def kernel(dayofweek, time, sex, age, month, day,
           W_dayofweek, W_time, W_sex, W_age, W_month, W_day):
    table = jnp.concatenate(
        [_pair_table(W_dayofweek, W_time),
         _pair_table(W_sex, W_age),
         _pair_table(W_month, W_day)],
        axis=0,
    )
    table_rep = jnp.broadcast_to(
        table[None], (_REPLICAS, TABLE_ROWS, PAIR_DIM)
    ).reshape(_REPLICAS * TABLE_ROWS, PAIR_DIM)
    idx = [a.astype(jnp.int32)
           for a in (dayofweek, time, sex, age, month, day)]
    return _embed_concat(*idx, table_rep)


# speedup vs baseline: 1.0507x; 1.0507x over previous
import functools

import jax
import jax.numpy as jnp
from jax import lax
from jax.experimental import pallas as pl
from jax.experimental.pallas import tpu as pltpu
from jax.experimental.pallas import tpu_sc as plsc

NUM_DIM = 64
BATCH = 16384
NUM_PAIRS = 3
PAIR_DIM = 2 * NUM_DIM
TABLE_ROWS = 7 * 24 + 2 * 100 + 12 * 31
_REPLICAS = 8

_NC = 2
_NS = 16
_NW = _NC * _NS
_S_PER_W = BATCH // _NW
_CHUNK = 128
_NBLOCK = _S_PER_W // _CHUNK
_IDX_PER_W = _S_PER_W * NUM_PAIRS

_mesh = plsc.VectorSubcoreMesh(core_axis_name="c", subcore_axis_name="s")


@functools.partial(
    pl.kernel,
    out_type=jax.ShapeDtypeStruct((BATCH, NUM_PAIRS * PAIR_DIM), jnp.float32),
    mesh=_mesh,
    scratch_types=[
        pltpu.VMEM((6 * _S_PER_W,), jnp.int32),
        pltpu.VMEM((_IDX_PER_W,), jnp.int32),
        pltpu.VMEM((_CHUNK, NUM_PAIRS * PAIR_DIM), jnp.float32),
        pltpu.VMEM((_CHUNK, NUM_PAIRS * PAIR_DIM), jnp.float32),
        pltpu.SemaphoreType.DMA,
        pltpu.SemaphoreType.DMA,
    ],
)
def _embed_concat(i0, i1, i2, i3, i4, i5, table_hbm, out, raw_v, idx_v,
                  buf_a, buf_b, gsem, osem):
    bufs = (buf_a, buf_b)
    sid = lax.axis_index("s")
    wid = sid * _NC + lax.axis_index("c")
    base = wid * _S_PER_W

    raw_loads = [
        pltpu.async_copy(ref.at[pl.ds(base, _S_PER_W)],
                         raw_v.at[pl.ds(f * _S_PER_W, _S_PER_W)], osem)
        for f, ref in enumerate((i0, i1, i2, i3, i4, i5))
    ]
    for ld in raw_loads:
        ld.wait()

    def build_block(b):
        for c, (mult, offset) in enumerate(((24, 0), (100, 168), (31, 368))):
            bias = offset + ((wid * _NBLOCK * NUM_PAIRS
                              + b * NUM_PAIRS + c) % _REPLICAS) * TABLE_ROWS
            for k in range(_CHUNK // 16):
                s = b * _CHUNK + k * 16
                va = raw_v[pl.ds(2 * c * _S_PER_W + s, 16)]
                vb = raw_v[pl.ds((2 * c + 1) * _S_PER_W + s, 16)]
                idx_v[pl.ds((b * NUM_PAIRS + c) * _CHUNK + k * 16, 16)] = (
                    va * mult + vb + bias)

    build_block(0)
    out_writes = []
    for b in range(_NBLOCK):
        buf = bufs[b % 2]
        if b >= 2:
            out_writes[b - 2].wait()
        gathers = [
            pltpu.async_copy(
                table_hbm.at[idx_v.at[pl.ds((b * NUM_PAIRS + c) * _CHUNK,
                                           _CHUNK)]],
                buf.at[:, pl.ds(c * PAIR_DIM, PAIR_DIM)],
                gsem,
            )
            for c in range(NUM_PAIRS)
        ]
        if b + 1 < _NBLOCK:
            build_block(b + 1)
        for g in gathers:
            g.wait()
        out_writes.append(
            pltpu.async_copy(
                buf,
                out.at[pl.ds(wid * _S_PER_W + b * _CHUNK, _CHUNK)],
                osem,
            )
        )
    out_writes[_NBLOCK - 2].wait()
    out_writes[_NBLOCK - 1].wait()


def _pair_table(wa, wb):
    va, vb = wa.shape[0], wb.shape[0]
    return jnp.concatenate(
        [jnp.broadcast_to(wa[:, None, :], (va, vb, NUM_DIM)),
         jnp.broadcast_to(wb[None, :, :], (va, vb, NUM_DIM))],
        axis=-1,
    ).reshape(va * vb, PAIR_DIM)


def kernel(dayofweek, time, sex, age, month, day,
           W_dayofweek, W_time, W_sex, W_age, W_month, W_day):
    table = jnp.concatenate(
        [_pair_table(W_dayofweek, W_time),
         _pair_table(W_sex, W_age),
         _pair_table(W_month, W_day)],
        axis=0,
    )
    table_rep = jnp.broadcast_to(
        table[None], (_REPLICAS, TABLE_ROWS, PAIR_DIM)
    ).reshape(_REPLICAS * TABLE_ROWS, PAIR_DIM)
    idx = [a.astype(jnp.int32)
           for a in (dayofweek, time, sex, age, month, day)]
    return _embed_concat(*idx, table_rep)
